# baseline (speedup 1.0000x reference)
import jax
import jax.numpy as jnp
from jax.experimental import pallas as pl

_DEPTH = 4
_K = 1024
_D = 256
_KC = 256


def _rq_kernel(x_ref, cb_ref, cbs_ref, out_ref, codes_ref, loss_ref):
    step = pl.program_id(0)

    @pl.when(step == 0)
    def _():
        loss_ref[...] = jnp.zeros((1, 1), jnp.float32)

    xb = x_ref[...]
    T = xb.shape[0]
    residual = xb
    agg = jnp.zeros_like(xb)
    loss_acc = jnp.zeros((), jnp.float32)
    code_cols = []
    n_chunks = _K // _KC
    for i in range(_DEPTH):
        norm = jnp.sqrt(jnp.sum(residual * residual, axis=1, keepdims=True))
        inp = residual / jnp.maximum(norm, 1e-12)
        in_sq = jnp.sum(inp * inp, axis=1, keepdims=True)
        inp_bf = inp.astype(jnp.bfloat16)

        best_val = jnp.full((T, 1), jnp.inf, jnp.float32)
        best_idx = jnp.zeros((T, 1), jnp.int32)
        for c in range(n_chunks):
            cb_c = cb_ref[i, c * _KC:(c + 1) * _KC, :]
            cb_sq = jnp.sum(cb_c * cb_c, axis=1)[None, :]
            ab = jax.lax.dot_general(
                inp_bf, cb_ref[i, c * _KC:(c + 1) * _KC, :].astype(jnp.bfloat16),
                (((1,), (1,)), ((), ())),
                preferred_element_type=jnp.float32)
            scores = in_sq + cb_sq - 2.0 * ab
            c_val = jnp.min(scores, axis=1, keepdims=True)
            c_idx = jnp.argmin(scores, axis=1)[:, None] + c * _KC
            take = c_val < best_val
            best_val = jnp.where(take, c_val, best_val)
            best_idx = jnp.where(take, c_idx, best_idx)

        quant = jnp.zeros((T, _D), jnp.float32)
        lane = jax.lax.broadcasted_iota(jnp.int32, (T, _KC), 1)
        for c in range(n_chunks):
            onehot = (lane + c * _KC == best_idx).astype(jnp.bfloat16)
            part = jnp.zeros((T, _D), jnp.float32)
            for p in range(3):
                part = part + jax.lax.dot_general(
                    onehot, cbs_ref[p, i, c * _KC:(c + 1) * _KC, :],
                    (((1,), (0,)), ((), ())),
                    preferred_element_type=jnp.float32)
            quant = quant + part

        residual = residual - quant
        agg = agg + quant
        diff = xb - agg
        loss_acc = loss_acc + jnp.sum(diff * diff)
        code_cols.append(best_idx)

    out_ref[...] = xb + (agg - xb)
    codes_ref[...] = jnp.concatenate(code_cols, axis=1)
    loss_ref[...] += jnp.reshape(loss_acc, (1, 1))


@jax.jit
def kernel(x, codebooks):
    orig_shape = x.shape
    N = x.shape[0] * x.shape[1] * x.shape[2]
    D = x.shape[3]
    flat = x.reshape(N, D)

    cb_hi = codebooks.astype(jnp.bfloat16)
    r1 = codebooks - cb_hi.astype(jnp.float32)
    cb_mid = r1.astype(jnp.bfloat16)
    r2 = r1 - cb_mid.astype(jnp.float32)
    cb_lo = r2.astype(jnp.bfloat16)
    cb_split = jnp.stack([cb_hi, cb_mid, cb_lo])

    T = 512
    grid = (N // T,)

    out, codes, loss = pl.pallas_call(
        _rq_kernel,
        grid=grid,
        in_specs=[
            pl.BlockSpec((T, D), lambda i: (i, 0)),
            pl.BlockSpec((_DEPTH, _K, D), lambda i: (0, 0, 0)),
            pl.BlockSpec((3, _DEPTH, _K, D), lambda i: (0, 0, 0, 0)),
        ],
        out_specs=[
            pl.BlockSpec((T, D), lambda i: (i, 0)),
            pl.BlockSpec((T, _DEPTH), lambda i: (i, 0)),
            pl.BlockSpec((1, 1), lambda i: (0, 0)),
        ],
        out_shape=[
            jax.ShapeDtypeStruct((N, D), jnp.float32),
            jax.ShapeDtypeStruct((N, _DEPTH), jnp.int32),
            jax.ShapeDtypeStruct((1, 1), jnp.float32),
        ],
    )(flat, codebooks, cb_split)

    quants = out.reshape(orig_shape)
    codes = codes.reshape(orig_shape[:-1] + (_DEPTH,))
    commitment_loss = loss[0, 0] / (N * D * _DEPTH)
    return quants, commitment_loss, codes
